# baseline (speedup 1.0000x reference)
import jax
import jax.numpy as jnp
from jax import lax
from jax.experimental import pallas as pl
from jax.experimental.pallas import tpu as pltpu
from jax.experimental.pallas import tpu_sc as plsc

SITES = 10000
IN_F = 128
OUT_F = 128
B = 8
R = SITES * IN_F
NNZ = 2000000
NNZP = 2048000
EDGE_TYPES = 8

NC = 2
NS = 16
NCHUNK = 8
CH = R // NCHUNK
CPS = NCHUNK // NC
EPT = NNZP // NS
BLK = 3200
NBLK = EPT // BLK
K = 2048
KC = 6144
GCH = 1024
GQ = KC // GCH
ROWS_PT = CH // NS
ZR = 1000
PAD_ROW = 1 << 30

_i32 = jnp.int32
_f32 = jnp.float32


_PC = 500
_PR = 16
_NREAL = NNZ // (_PR * _PC)
_NTOT = NNZP // (_PR * _PC)


def _prep_body(w_ref, r_ref, c_ref, e_ref, rp_ref, cp_ref, wp_ref):
    i = pl.program_id(0)

    @pl.when(i < _NREAL)
    def _():
        rp_ref[...] = r_ref[0]
        cp_ref[...] = c_ref[0]
        et = e_ref[...]
        wv = jnp.zeros((_PR, _PC), _f32)
        for t in range(EDGE_TYPES):
            wv = wv + jnp.where(et == t, w_ref[t], 0.0)
        wp_ref[...] = wv

    @pl.when(i >= _NREAL)
    def _():
        rp_ref[...] = jnp.full((_PR, _PC), PAD_ROW, _i32)
        cp_ref[...] = jnp.zeros((_PR, _PC), _i32)
        wp_ref[...] = jnp.zeros((_PR, _PC), _f32)


def _prep(weight, signal3, et2):
    sh_i = jax.ShapeDtypeStruct((NNZP // _PC, _PC), _i32)
    sh_f = jax.ShapeDtypeStruct((NNZP // _PC, _PC), _f32)
    clamp = lambda i: (jnp.minimum(i, _NREAL - 1), 0)
    return pl.pallas_call(
        _prep_body,
        grid=(_NTOT,),
        in_specs=[
            pl.BlockSpec(memory_space=pltpu.SMEM),
            pl.BlockSpec((1, _PR, _PC), lambda i: (0, jnp.minimum(i, _NREAL - 1), 0)),
            pl.BlockSpec((1, _PR, _PC), lambda i: (1, jnp.minimum(i, _NREAL - 1), 0)),
            pl.BlockSpec((_PR, _PC), clamp),
        ],
        out_specs=[
            pl.BlockSpec((_PR, _PC), lambda i: (i, 0)),
            pl.BlockSpec((_PR, _PC), lambda i: (i, 0)),
            pl.BlockSpec((_PR, _PC), lambda i: (i, 0)),
        ],
        out_shape=[sh_i, sh_i, sh_f],
    )(weight, signal3, signal3, et2)


def _sc_body(*refs):
    xps = refs[0:B]
    rows, cols, wvs = refs[B:B + 3]
    outs = refs[B + 3:2 * B + 3]
    (er0, ec0, ew0, er1, ec1, ew1, cbuf, rbuf, wbuf,
     xgA, xgB, zbuf) = refs[2 * B + 3:2 * B + 15]
    accs = refs[2 * B + 15:3 * B + 15]
    esemA, esemB, gsem, ssem = refs[3 * B + 15:]

    c = lax.axis_index("c")
    s = lax.axis_index("s")
    zerov_f = jnp.zeros((16,), _f32)
    zerov_i = jnp.zeros((16,), _i32)

    def _init(g, _):
        sl = pl.ds(g * 16, 16)
        cbuf[sl] = zerov_i
        rbuf[sl] = zerov_i
        wbuf[sl] = zerov_f
        return 0

    lax.fori_loop(0, KC // 16, _init, 0)

    def _initz(g, _):
        zbuf[pl.ds(g * 16, 16)] = zerov_f
        return 0

    lax.fori_loop(0, ZR // 16, _initz, 0)

    def _fire(base, er, ec, ew, sem):
        pltpu.async_copy(rows.at[pl.ds(base, BLK)], er, sem)
        pltpu.async_copy(cols.at[pl.ds(base, BLK)], ec, sem)
        pltpu.async_copy(wvs.at[pl.ds(base, BLK)], ew, sem)

    def _wait(er, ec, ew, sem):
        pltpu.make_async_copy(rows.at[pl.ds(0, BLK)], er, sem).wait()
        pltpu.make_async_copy(cols.at[pl.ds(0, BLK)], ec, sem).wait()
        pltpu.make_async_copy(wvs.at[pl.ds(0, BLK)], ew, sem).wait()

    def _gather_plane(b, xg, ng):
        def _fg(g, _):
            pltpu.async_copy(xps[b].at[cbuf.at[pl.ds(g * GCH, GCH)]],
                             xg.at[pl.ds(g * GCH, GCH)], gsem)
            return 0

        lax.fori_loop(0, ng, _fg, 0)

    def _drain_gather(xg, ng):
        def _dg(g, _):
            pltpu.make_async_copy(rows.at[pl.ds(0, GCH)],
                                  xg.at[pl.ds(g * GCH, GCH)], gsem).wait()
            return 0

        lax.fori_loop(0, ng, _dg, 0)

    def _scale_scatter(b, xg, ng):
        def _sc(v, _):
            sl = pl.ds(v * 16, 16)
            xg[sl] = xg[sl] * wbuf[sl]
            return 0

        lax.fori_loop(0, ng * (GCH // 16), _sc, 0)

        def _fs(g, _):
            pltpu.async_copy(xg.at[pl.ds(g * GCH, GCH)],
                             accs[b].at[rbuf.at[pl.ds(g * GCH, GCH)]],
                             ssem, add=True)
            return 0

        lax.fori_loop(0, ng, _fs, 0)

    def _drain_scatter(xg, ng):
        def _ds(g, _):
            pltpu.make_async_copy(rows.at[pl.ds(0, GCH)],
                                  xg.at[pl.ds(g * GCH, GCH)], ssem).wait()
            return 0

        lax.fori_loop(0, ng, _ds, 0)

    def _flush(ng):
        _gather_plane(0, xgA, ng)
        for b in range(B):
            xg, xo = (xgA, xgB) if b % 2 == 0 else (xgB, xgA)
            _drain_gather(xg, ng)
            if b + 1 < B:
                if b >= 1:
                    _drain_scatter(xo, ng)
                _gather_plane(b + 1, xo, ng)
            _scale_scatter(b, xg, ng)
        _drain_scatter(xgA, ng)
        _drain_scatter(xgB, ng)

        def _zw(g, _):
            wbuf[pl.ds(g * 16, 16)] = zerov_f
            return 0

        lax.fori_loop(0, ng * (GCH // 16), _zw, 0)

    def _chunk(j, _):
        lo = (c * CPS + j) * CH

        for b in range(B):
            for z in range(ROWS_PT // ZR):
                pltpu.sync_copy(
                    zbuf, accs[b].at[pl.ds(s * ROWS_PT + z * ZR, ZR)])
        plsc.subcore_barrier()

        def _process(er, ec, ew, ptr0):
            def _grp4(g, ptr):
                for u in range(4):
                    sl = pl.ds(g * 64 + u * 16, 16)
                    local = er[sl] - lo
                    m = local.astype(jnp.uint32) < jnp.uint32(CH)
                    dsl = pl.ds(ptr, 16)
                    plsc.store_compressed(cbuf.at[dsl], ec[sl], mask=m)
                    plsc.store_compressed(rbuf.at[dsl], local, mask=m)
                    plsc.store_compressed(wbuf.at[dsl], ew[sl], mask=m)
                    ptr = ptr + jnp.sum(m.astype(_i32))
                return ptr

            p1 = lax.fori_loop(0, BLK // 64, _grp4, ptr0)
            full = p1 > K
            pl.when(full)(lambda: _flush((p1 + GCH - 1) // GCH))
            return jnp.where(full, 0, p1)

        ebase = s * EPT
        _fire(ebase, er0, ec0, ew0, esemA)

        def _blkpair(b2, ptr):
            _fire(ebase + (2 * b2 + 1) * BLK, er1, ec1, ew1, esemB)
            _wait(er0, ec0, ew0, esemA)
            ptr = _process(er0, ec0, ew0, ptr)
            _fire(ebase + ((2 * b2 + 2) % NBLK) * BLK, er0, ec0, ew0,
                  esemA)
            _wait(er1, ec1, ew1, esemB)
            ptr = _process(er1, ec1, ew1, ptr)
            return ptr

        ptr_end = lax.fori_loop(0, NBLK // 2, _blkpair, 0)
        _wait(er0, ec0, ew0, esemA)
        _flush((ptr_end + GCH - 1) // GCH)

        plsc.subcore_barrier()
        for b in range(B):
            pltpu.sync_copy(accs[b].at[pl.ds(s * ROWS_PT, ROWS_PT)],
                            outs[b].at[pl.ds(lo + s * ROWS_PT, ROWS_PT)])
        return 0

    lax.fori_loop(0, CPS, _chunk, 0)


def _sc_call(xplanes, rows, cols, wvs):
    mesh = plsc.VectorSubcoreMesh(core_axis_name="c", subcore_axis_name="s")
    kern = pl.kernel(
        _sc_body,
        out_type=[jax.ShapeDtypeStruct((R,), _f32) for _ in range(B)],
        mesh=mesh,
        scratch_types=[
            pltpu.VMEM((BLK,), _i32), pltpu.VMEM((BLK,), _i32),
            pltpu.VMEM((BLK,), _f32),
            pltpu.VMEM((BLK,), _i32), pltpu.VMEM((BLK,), _i32),
            pltpu.VMEM((BLK,), _f32),
            pltpu.VMEM((KC,), _i32),
            pltpu.VMEM((KC,), _i32),
            pltpu.VMEM((KC,), _f32),
            pltpu.VMEM((KC,), _f32),
            pltpu.VMEM((KC,), _f32),
            pltpu.VMEM((ZR,), _f32),
        ] + [pltpu.VMEM_SHARED((CH,), _f32) for _ in range(B)] + [
            pltpu.SemaphoreType.DMA, pltpu.SemaphoreType.DMA,
            pltpu.SemaphoreType.DMA, pltpu.SemaphoreType.DMA,
        ],
        compiler_params=pltpu.CompilerParams(needs_layout_passes=False,
                                             use_tc_tiling_on_sc=False),
    )
    return kern(*xplanes, rows, cols, wvs)


@jax.jit
def kernel(input, signal, edge_type, weight):
    inp2 = input.reshape(B, R)
    xplanes = [inp2[b] for b in range(B)]
    signal3 = signal.reshape(2, NNZ // _PC, _PC)
    et2 = edge_type.reshape(NNZ // _PC, _PC)
    rp, cp, wp = _prep(weight, signal3, et2)
    outs = _sc_call(xplanes, rp.reshape(-1), cp.reshape(-1), wp.reshape(-1))
    y = jnp.stack(outs, axis=0)
    return y.reshape(B, SITES, OUT_F)


# speedup vs baseline: 2.5687x; 2.5687x over previous
import jax
import jax.numpy as jnp
from jax import lax
from jax.experimental import pallas as pl
from jax.experimental.pallas import tpu as pltpu
from jax.experimental.pallas import tpu_sc as plsc

SITES = 10000
IN_F = 128
OUT_F = 128
B = 8
R = SITES * IN_F
NNZ = 2000000
NNZP = 2048000
EDGE_TYPES = 8

NC = 2
NS = 16
NCHUNK = 8
CH = R // NCHUNK
CPS = NCHUNK // NC
EPT = NNZP // NS
BLK = 4000
NBLK = EPT // BLK
K = 4096
KC = K
GCH = 1024
GQ = KC // GCH
ROWS_PT = CH // NS
ZR = 2000
PAD_ROW = 1 << 30

_i32 = jnp.int32
_f32 = jnp.float32


_PC = 500
_PR = 16
_NREAL = NNZ // (_PR * _PC)
_NTOT = NNZP // (_PR * _PC)


def _prep_body(w_ref, r_ref, c_ref, e_ref, rp_ref, cp_ref, wp_ref):
    i = pl.program_id(0)

    @pl.when(i < _NREAL)
    def _():
        rp_ref[...] = r_ref[0]
        cp_ref[...] = c_ref[0]
        et = e_ref[...]
        wv = jnp.zeros((_PR, _PC), _f32)
        for t in range(EDGE_TYPES):
            wv = wv + jnp.where(et == t, w_ref[t], 0.0)
        wp_ref[...] = wv

    @pl.when(i >= _NREAL)
    def _():
        rp_ref[...] = jnp.full((_PR, _PC), PAD_ROW, _i32)
        cp_ref[...] = jnp.zeros((_PR, _PC), _i32)
        wp_ref[...] = jnp.zeros((_PR, _PC), _f32)


def _prep(weight, signal3, et2):
    sh_i = jax.ShapeDtypeStruct((NNZP // _PC, _PC), _i32)
    sh_f = jax.ShapeDtypeStruct((NNZP // _PC, _PC), _f32)
    clamp = lambda i: (jnp.minimum(i, _NREAL - 1), 0)
    return pl.pallas_call(
        _prep_body,
        grid=(_NTOT,),
        in_specs=[
            pl.BlockSpec(memory_space=pltpu.SMEM),
            pl.BlockSpec((1, _PR, _PC), lambda i: (0, jnp.minimum(i, _NREAL - 1), 0)),
            pl.BlockSpec((1, _PR, _PC), lambda i: (1, jnp.minimum(i, _NREAL - 1), 0)),
            pl.BlockSpec((_PR, _PC), clamp),
        ],
        out_specs=[
            pl.BlockSpec((_PR, _PC), lambda i: (i, 0)),
            pl.BlockSpec((_PR, _PC), lambda i: (i, 0)),
            pl.BlockSpec((_PR, _PC), lambda i: (i, 0)),
        ],
        out_shape=[sh_i, sh_i, sh_f],
    )(weight, signal3, signal3, et2)


def _sc_body(*refs):
    xps = refs[0:B]
    rows, cols, wvs = refs[B:B + 3]
    outs = refs[B + 3:2 * B + 3]
    (er0, ec0, ew0, er1, ec1, ew1, cbuf, rbuf, wbuf,
     xgA, xgB, zbuf) = refs[2 * B + 3:2 * B + 15]
    accs = refs[2 * B + 15:3 * B + 15]
    esemA, esemB, gsem, ssem = refs[3 * B + 15:]

    c = lax.axis_index("c")
    s = lax.axis_index("s")
    zerov_f = jnp.zeros((16,), _f32)
    zerov_i = jnp.zeros((16,), _i32)

    def _init(g, _):
        sl = pl.ds(g * 16, 16)
        cbuf[sl] = zerov_i
        rbuf[sl] = zerov_i
        wbuf[sl] = zerov_f
        return 0

    lax.fori_loop(0, KC // 16, _init, 0)

    def _initz(g, _):
        zbuf[pl.ds(g * 16, 16)] = zerov_f
        return 0

    lax.fori_loop(0, ZR // 16, _initz, 0)

    def _fire(base, er, ec, ew, sem):
        pltpu.async_copy(rows.at[pl.ds(base, BLK)], er, sem)
        pltpu.async_copy(cols.at[pl.ds(base, BLK)], ec, sem)
        pltpu.async_copy(wvs.at[pl.ds(base, BLK)], ew, sem)

    def _wait(er, ec, ew, sem):
        pltpu.make_async_copy(rows.at[pl.ds(0, BLK)], er, sem).wait()
        pltpu.make_async_copy(cols.at[pl.ds(0, BLK)], ec, sem).wait()
        pltpu.make_async_copy(wvs.at[pl.ds(0, BLK)], ew, sem).wait()

    def _gather_plane(b, xg, ng):
        def _fg(g, _):
            pltpu.async_copy(xps[b].at[cbuf.at[pl.ds(g * GCH, GCH)]],
                             xg.at[pl.ds(g * GCH, GCH)], gsem)
            return 0

        lax.fori_loop(0, ng, _fg, 0)

    def _drain_gather(xg, ng):
        def _dg(g, _):
            pltpu.make_async_copy(rows.at[pl.ds(0, GCH)],
                                  xg.at[pl.ds(g * GCH, GCH)], gsem).wait()
            return 0

        lax.fori_loop(0, ng, _dg, 0)

    def _scale_scatter(b, xg, ng):
        def _sc(v, _):
            sl = pl.ds(v * 16, 16)
            xg[sl] = xg[sl] * wbuf[sl]
            return 0

        lax.fori_loop(0, ng * (GCH // 16), _sc, 0)

        def _fs(g, _):
            pltpu.async_copy(xg.at[pl.ds(g * GCH, GCH)],
                             accs[b].at[rbuf.at[pl.ds(g * GCH, GCH)]],
                             ssem, add=True)
            return 0

        lax.fori_loop(0, ng, _fs, 0)

    def _drain_scatter(xg, ng):
        def _ds(g, _):
            pltpu.make_async_copy(rows.at[pl.ds(0, GCH)],
                                  xg.at[pl.ds(g * GCH, GCH)], ssem).wait()
            return 0

        lax.fori_loop(0, ng, _ds, 0)

    def _flush(ng):
        _gather_plane(0, xgA, ng)
        for b in range(B):
            xg, xo = (xgA, xgB) if b % 2 == 0 else (xgB, xgA)
            _drain_gather(xg, ng)
            if b + 1 < B:
                if b >= 1:
                    _drain_scatter(xo, ng)
                _gather_plane(b + 1, xo, ng)
            _scale_scatter(b, xg, ng)
        _drain_scatter(xgA, ng)
        _drain_scatter(xgB, ng)

        def _zw(g, _):
            wbuf[pl.ds(g * 16, 16)] = zerov_f
            return 0

        lax.fori_loop(0, ng * (GCH // 16), _zw, 0)

    def _chunk(j, _):
        lo = (c * CPS + j) * CH

        for b in range(B):
            for z in range(ROWS_PT // ZR):
                pltpu.sync_copy(
                    zbuf, accs[b].at[pl.ds(s * ROWS_PT + z * ZR, ZR)])
        plsc.subcore_barrier()

        def _process(er, ec, ew, ptr0):
            def _grp(g, ptr):
                sl = pl.ds(g * 16, 16)
                local = er[sl] - lo
                m = local.astype(jnp.uint32) < jnp.uint32(CH)
                dsl = pl.ds(ptr, 16)
                plsc.store_compressed(cbuf.at[dsl], ec[sl], mask=m)
                plsc.store_compressed(rbuf.at[dsl], local, mask=m)
                plsc.store_compressed(wbuf.at[dsl], ew[sl], mask=m)
                p2 = ptr + jnp.sum(m.astype(_i32))
                full = p2 > K - 16
                pl.when(full)(lambda: _flush(jnp.int32(GQ)))
                return jnp.where(full, 0, p2)

            return lax.fori_loop(0, BLK // 16, _grp, ptr0)

        ebase = s * EPT
        _fire(ebase, er0, ec0, ew0, esemA)

        def _blkpair(b2, ptr):
            _fire(ebase + (2 * b2 + 1) * BLK, er1, ec1, ew1, esemB)
            _wait(er0, ec0, ew0, esemA)
            ptr = _process(er0, ec0, ew0, ptr)
            _fire(ebase + ((2 * b2 + 2) % NBLK) * BLK, er0, ec0, ew0,
                  esemA)
            _wait(er1, ec1, ew1, esemB)
            ptr = _process(er1, ec1, ew1, ptr)
            return ptr

        ptr_end = lax.fori_loop(0, NBLK // 2, _blkpair, 0)
        _wait(er0, ec0, ew0, esemA)
        _flush((ptr_end + GCH - 1) // GCH)

        plsc.subcore_barrier()
        for b in range(B):
            pltpu.sync_copy(accs[b].at[pl.ds(s * ROWS_PT, ROWS_PT)],
                            outs[b].at[pl.ds(lo + s * ROWS_PT, ROWS_PT)])
        return 0

    lax.fori_loop(0, CPS, _chunk, 0)


def _sc_call(xplanes, rows, cols, wvs):
    mesh = plsc.VectorSubcoreMesh(core_axis_name="c", subcore_axis_name="s")
    kern = pl.kernel(
        _sc_body,
        out_type=[jax.ShapeDtypeStruct((R,), _f32) for _ in range(B)],
        mesh=mesh,
        scratch_types=[
            pltpu.VMEM((BLK,), _i32), pltpu.VMEM((BLK,), _i32),
            pltpu.VMEM((BLK,), _f32),
            pltpu.VMEM((BLK,), _i32), pltpu.VMEM((BLK,), _i32),
            pltpu.VMEM((BLK,), _f32),
            pltpu.VMEM((KC,), _i32),
            pltpu.VMEM((KC,), _i32),
            pltpu.VMEM((KC,), _f32),
            pltpu.VMEM((KC,), _f32),
            pltpu.VMEM((KC,), _f32),
            pltpu.VMEM((ZR,), _f32),
        ] + [pltpu.VMEM_SHARED((CH,), _f32) for _ in range(B)] + [
            pltpu.SemaphoreType.DMA, pltpu.SemaphoreType.DMA,
            pltpu.SemaphoreType.DMA, pltpu.SemaphoreType.DMA,
        ],
        compiler_params=pltpu.CompilerParams(needs_layout_passes=False,
                                             use_tc_tiling_on_sc=False),
    )
    return kern(*xplanes, rows, cols, wvs)


@jax.jit
def kernel(input, signal, edge_type, weight):
    inp2 = input.reshape(B, R)
    xplanes = [inp2[b] for b in range(B)]
    signal3 = signal.reshape(2, NNZ // _PC, _PC)
    et2 = edge_type.reshape(NNZ // _PC, _PC)
    rp, cp, wp = _prep(weight, signal3, et2)
    outs = _sc_call(xplanes, rp.reshape(-1), cp.reshape(-1), wp.reshape(-1))
    y = jnp.stack(outs, axis=0)
    return y.reshape(B, SITES, OUT_F)


# speedup vs baseline: 2.7113x; 1.0555x over previous
import jax
import jax.numpy as jnp
from jax import lax
from jax.experimental import pallas as pl
from jax.experimental.pallas import tpu as pltpu
from jax.experimental.pallas import tpu_sc as plsc

SITES = 10000
IN_F = 128
OUT_F = 128
B = 8
R = SITES * IN_F
NNZ = 2000000
NNZP = 2048000
EDGE_TYPES = 8

NC = 2
NS = 16
NCHUNK = 8
CH = R // NCHUNK
CPS = NCHUNK // NC
EPT = NNZP // NS
BLK = 4000
NBLK = EPT // BLK
K = 4096
KC = K
GCH = 1024
GQ = KC // GCH
ROWS_PT = CH // NS
ZR = 2000
PAD_ROW = 1 << 30

_i32 = jnp.int32
_f32 = jnp.float32


_PC = 500
_PR = 16
_NREAL = NNZ // (_PR * _PC)
_NTOT = NNZP // (_PR * _PC)


def _prep_body(w_ref, r_ref, c_ref, e_ref, rp_ref, cp_ref, wp_ref):
    i = pl.program_id(0)

    @pl.when(i < _NREAL)
    def _():
        rp_ref[...] = r_ref[0]
        cp_ref[...] = c_ref[0]
        et = e_ref[...]
        wv = jnp.zeros((_PR, _PC), _f32)
        for t in range(EDGE_TYPES):
            wv = wv + jnp.where(et == t, w_ref[t], 0.0)
        wp_ref[...] = wv

    @pl.when(i >= _NREAL)
    def _():
        rp_ref[...] = jnp.full((_PR, _PC), PAD_ROW, _i32)
        cp_ref[...] = jnp.zeros((_PR, _PC), _i32)
        wp_ref[...] = jnp.zeros((_PR, _PC), _f32)


def _prep(weight, signal3, et2):
    sh_i = jax.ShapeDtypeStruct((NNZP // _PC, _PC), _i32)
    sh_f = jax.ShapeDtypeStruct((NNZP // _PC, _PC), _f32)
    clamp = lambda i: (jnp.minimum(i, _NREAL - 1), 0)
    return pl.pallas_call(
        _prep_body,
        grid=(_NTOT,),
        in_specs=[
            pl.BlockSpec(memory_space=pltpu.SMEM),
            pl.BlockSpec((1, _PR, _PC), lambda i: (0, jnp.minimum(i, _NREAL - 1), 0)),
            pl.BlockSpec((1, _PR, _PC), lambda i: (1, jnp.minimum(i, _NREAL - 1), 0)),
            pl.BlockSpec((_PR, _PC), clamp),
        ],
        out_specs=[
            pl.BlockSpec((_PR, _PC), lambda i: (i, 0)),
            pl.BlockSpec((_PR, _PC), lambda i: (i, 0)),
            pl.BlockSpec((_PR, _PC), lambda i: (i, 0)),
        ],
        out_shape=[sh_i, sh_i, sh_f],
    )(weight, signal3, signal3, et2)


def _sc_body(*refs):
    xps = refs[0:B]
    rows, cols, wvs = refs[B:B + 3]
    outs = refs[B + 3:2 * B + 3]
    (er0, ec0, ew0, er1, ec1, ew1, cbuf, rbuf, wbuf,
     xgA, xgB, zbuf) = refs[2 * B + 3:2 * B + 15]
    accs = refs[2 * B + 15:3 * B + 15]
    esemA, esemB, gsem, ssem = refs[3 * B + 15:]

    c = lax.axis_index("c")
    s = lax.axis_index("s")
    zerov_f = jnp.zeros((16,), _f32)
    zerov_i = jnp.zeros((16,), _i32)

    def _init(g, _):
        sl = pl.ds(g * 16, 16)
        cbuf[sl] = zerov_i
        rbuf[sl] = zerov_i
        wbuf[sl] = zerov_f
        return 0

    lax.fori_loop(0, KC // 16, _init, 0)

    def _initz(g, _):
        zbuf[pl.ds(g * 16, 16)] = zerov_f
        return 0

    lax.fori_loop(0, ZR // 16, _initz, 0)

    def _fire(base, er, ec, ew, sem):
        pltpu.async_copy(rows.at[pl.ds(base, BLK)], er, sem)
        pltpu.async_copy(cols.at[pl.ds(base, BLK)], ec, sem)
        pltpu.async_copy(wvs.at[pl.ds(base, BLK)], ew, sem)

    def _wait(er, ec, ew, sem):
        pltpu.make_async_copy(rows.at[pl.ds(0, BLK)], er, sem).wait()
        pltpu.make_async_copy(cols.at[pl.ds(0, BLK)], ec, sem).wait()
        pltpu.make_async_copy(wvs.at[pl.ds(0, BLK)], ew, sem).wait()

    def _gather_plane(b, xg, ng):
        def _fg(g, _):
            pltpu.async_copy(xps[b].at[cbuf.at[pl.ds(g * GCH, GCH)]],
                             xg.at[pl.ds(g * GCH, GCH)], gsem)
            return 0

        lax.fori_loop(0, ng, _fg, 0)

    def _drain_gather(xg, ng):
        def _dg(g, _):
            pltpu.make_async_copy(rows.at[pl.ds(0, GCH)],
                                  xg.at[pl.ds(g * GCH, GCH)], gsem).wait()
            return 0

        lax.fori_loop(0, ng, _dg, 0)

    def _scale_scatter(b, xg, ng):
        def _sc(v, _):
            sl = pl.ds(v * 16, 16)
            xg[sl] = xg[sl] * wbuf[sl]
            return 0

        lax.fori_loop(0, ng * (GCH // 16), _sc, 0)

        def _fs(g, _):
            pltpu.async_copy(xg.at[pl.ds(g * GCH, GCH)],
                             accs[b].at[rbuf.at[pl.ds(g * GCH, GCH)]],
                             ssem, add=True)
            return 0

        lax.fori_loop(0, ng, _fs, 0)

    def _drain_scatter(xg, ng):
        def _ds(g, _):
            pltpu.make_async_copy(rows.at[pl.ds(0, GCH)],
                                  xg.at[pl.ds(g * GCH, GCH)], ssem).wait()
            return 0

        lax.fori_loop(0, ng, _ds, 0)

    def _flush(ng):
        _gather_plane(0, xgA, ng)
        for b in range(B):
            xg, xo = (xgA, xgB) if b % 2 == 0 else (xgB, xgA)
            _drain_gather(xg, ng)
            if b + 1 < B:
                if b >= 1:
                    _drain_scatter(xo, ng)
                _gather_plane(b + 1, xo, ng)
            _scale_scatter(b, xg, ng)
        _drain_scatter(xgA, ng)
        _drain_scatter(xgB, ng)

        def _zw(g, _):
            wbuf[pl.ds(g * 16, 16)] = zerov_f
            return 0

        lax.fori_loop(0, ng * (GCH // 16), _zw, 0)

    def _chunk(j, _):
        lo = (c * CPS + j) * CH

        for b in range(B):
            for z in range(ROWS_PT // ZR):
                pltpu.sync_copy(
                    zbuf, accs[b].at[pl.ds(s * ROWS_PT + z * ZR, ZR)])
        plsc.subcore_barrier()

        def _process(er, ec, ew, ptr0):
            def _grp(g, ptr):
                sl = pl.ds(g * 16, 16)
                local = er[sl] - lo
                m = local.astype(jnp.uint32) < jnp.uint32(CH)
                dsl = pl.ds(ptr, 16)
                plsc.store_compressed(cbuf.at[dsl], ec[sl], mask=m)
                plsc.store_compressed(rbuf.at[dsl], local, mask=m)
                plsc.store_compressed(wbuf.at[dsl], ew[sl], mask=m)
                p2 = ptr + plsc.all_reduce_population_count(m)[0]
                full = p2 > K - 16
                pl.when(full)(lambda: _flush(jnp.int32(GQ)))
                return jnp.where(full, 0, p2)

            return lax.fori_loop(0, BLK // 16, _grp, ptr0)

        ebase = s * EPT
        _fire(ebase, er0, ec0, ew0, esemA)

        def _blkpair(b2, ptr):
            _fire(ebase + (2 * b2 + 1) * BLK, er1, ec1, ew1, esemB)
            _wait(er0, ec0, ew0, esemA)
            ptr = _process(er0, ec0, ew0, ptr)
            _fire(ebase + ((2 * b2 + 2) % NBLK) * BLK, er0, ec0, ew0,
                  esemA)
            _wait(er1, ec1, ew1, esemB)
            ptr = _process(er1, ec1, ew1, ptr)
            return ptr

        ptr_end = lax.fori_loop(0, NBLK // 2, _blkpair, 0)
        _wait(er0, ec0, ew0, esemA)
        _flush((ptr_end + GCH - 1) // GCH)

        plsc.subcore_barrier()
        for b in range(B):
            pltpu.sync_copy(accs[b].at[pl.ds(s * ROWS_PT, ROWS_PT)],
                            outs[b].at[pl.ds(lo + s * ROWS_PT, ROWS_PT)])
        return 0

    lax.fori_loop(0, CPS, _chunk, 0)


def _sc_call(xplanes, rows, cols, wvs):
    mesh = plsc.VectorSubcoreMesh(core_axis_name="c", subcore_axis_name="s")
    kern = pl.kernel(
        _sc_body,
        out_type=[jax.ShapeDtypeStruct((R,), _f32) for _ in range(B)],
        mesh=mesh,
        scratch_types=[
            pltpu.VMEM((BLK,), _i32), pltpu.VMEM((BLK,), _i32),
            pltpu.VMEM((BLK,), _f32),
            pltpu.VMEM((BLK,), _i32), pltpu.VMEM((BLK,), _i32),
            pltpu.VMEM((BLK,), _f32),
            pltpu.VMEM((KC,), _i32),
            pltpu.VMEM((KC,), _i32),
            pltpu.VMEM((KC,), _f32),
            pltpu.VMEM((KC,), _f32),
            pltpu.VMEM((KC,), _f32),
            pltpu.VMEM((ZR,), _f32),
        ] + [pltpu.VMEM_SHARED((CH,), _f32) for _ in range(B)] + [
            pltpu.SemaphoreType.DMA, pltpu.SemaphoreType.DMA,
            pltpu.SemaphoreType.DMA, pltpu.SemaphoreType.DMA,
        ],
        compiler_params=pltpu.CompilerParams(needs_layout_passes=False,
                                             use_tc_tiling_on_sc=False),
    )
    return kern(*xplanes, rows, cols, wvs)


@jax.jit
def kernel(input, signal, edge_type, weight):
    inp2 = input.reshape(B, R)
    xplanes = [inp2[b] for b in range(B)]
    signal3 = signal.reshape(2, NNZ // _PC, _PC)
    et2 = edge_type.reshape(NNZ // _PC, _PC)
    rp, cp, wp = _prep(weight, signal3, et2)
    outs = _sc_call(xplanes, rp.reshape(-1), cp.reshape(-1), wp.reshape(-1))
    y = jnp.stack(outs, axis=0)
    return y.reshape(B, SITES, OUT_F)


# speedup vs baseline: 2.7145x; 1.0012x over previous
import jax
import jax.numpy as jnp
from jax import lax
from jax.experimental import pallas as pl
from jax.experimental.pallas import tpu as pltpu
from jax.experimental.pallas import tpu_sc as plsc

SITES = 10000
IN_F = 128
OUT_F = 128
B = 8
R = SITES * IN_F
NNZ = 2000000
NNZP = 2048000
EDGE_TYPES = 8

NC = 2
NS = 16
NCHUNK = 8
CH = R // NCHUNK
CPS = NCHUNK // NC
EPT = NNZP // NS
BLK = 4000
NBLK = EPT // BLK
K = 4096
KC = K
GCH = 4096
GQ = KC // GCH
ROWS_PT = CH // NS
ZR = 2000
PAD_ROW = 1 << 30

_i32 = jnp.int32
_f32 = jnp.float32


_PC = 500
_PR = 16
_NREAL = NNZ // (_PR * _PC)
_NTOT = NNZP // (_PR * _PC)


def _prep_body(w_ref, r_ref, c_ref, e_ref, rp_ref, cp_ref, wp_ref):
    i = pl.program_id(0)

    @pl.when(i < _NREAL)
    def _():
        rp_ref[...] = r_ref[0]
        cp_ref[...] = c_ref[0]
        et = e_ref[...]
        wv = jnp.zeros((_PR, _PC), _f32)
        for t in range(EDGE_TYPES):
            wv = wv + jnp.where(et == t, w_ref[t], 0.0)
        wp_ref[...] = wv

    @pl.when(i >= _NREAL)
    def _():
        rp_ref[...] = jnp.full((_PR, _PC), PAD_ROW, _i32)
        cp_ref[...] = jnp.zeros((_PR, _PC), _i32)
        wp_ref[...] = jnp.zeros((_PR, _PC), _f32)


def _prep(weight, signal3, et2):
    sh_i = jax.ShapeDtypeStruct((NNZP // _PC, _PC), _i32)
    sh_f = jax.ShapeDtypeStruct((NNZP // _PC, _PC), _f32)
    clamp = lambda i: (jnp.minimum(i, _NREAL - 1), 0)
    return pl.pallas_call(
        _prep_body,
        grid=(_NTOT,),
        in_specs=[
            pl.BlockSpec(memory_space=pltpu.SMEM),
            pl.BlockSpec((1, _PR, _PC), lambda i: (0, jnp.minimum(i, _NREAL - 1), 0)),
            pl.BlockSpec((1, _PR, _PC), lambda i: (1, jnp.minimum(i, _NREAL - 1), 0)),
            pl.BlockSpec((_PR, _PC), clamp),
        ],
        out_specs=[
            pl.BlockSpec((_PR, _PC), lambda i: (i, 0)),
            pl.BlockSpec((_PR, _PC), lambda i: (i, 0)),
            pl.BlockSpec((_PR, _PC), lambda i: (i, 0)),
        ],
        out_shape=[sh_i, sh_i, sh_f],
    )(weight, signal3, signal3, et2)


def _sc_body(*refs):
    xps = refs[0:B]
    rows, cols, wvs = refs[B:B + 3]
    outs = refs[B + 3:2 * B + 3]
    (er0, ec0, ew0, er1, ec1, ew1, cbuf, rbuf, wbuf,
     xgA, xgB, zbuf) = refs[2 * B + 3:2 * B + 15]
    accs = refs[2 * B + 15:3 * B + 15]
    esemA, esemB, gsem, ssem = refs[3 * B + 15:]

    c = lax.axis_index("c")
    s = lax.axis_index("s")
    zerov_f = jnp.zeros((16,), _f32)
    zerov_i = jnp.zeros((16,), _i32)

    def _init(g, _):
        sl = pl.ds(g * 16, 16)
        cbuf[sl] = zerov_i
        rbuf[sl] = zerov_i
        wbuf[sl] = zerov_f
        return 0

    lax.fori_loop(0, KC // 16, _init, 0)

    def _initz(g, _):
        zbuf[pl.ds(g * 16, 16)] = zerov_f
        return 0

    lax.fori_loop(0, ZR // 16, _initz, 0)

    def _fire(base, er, ec, ew, sem):
        pltpu.async_copy(rows.at[pl.ds(base, BLK)], er, sem)
        pltpu.async_copy(cols.at[pl.ds(base, BLK)], ec, sem)
        pltpu.async_copy(wvs.at[pl.ds(base, BLK)], ew, sem)

    def _wait(er, ec, ew, sem):
        pltpu.make_async_copy(rows.at[pl.ds(0, BLK)], er, sem).wait()
        pltpu.make_async_copy(cols.at[pl.ds(0, BLK)], ec, sem).wait()
        pltpu.make_async_copy(wvs.at[pl.ds(0, BLK)], ew, sem).wait()

    def _gather_plane(b, xg, ng):
        def _fg(g, _):
            pltpu.async_copy(xps[b].at[cbuf.at[pl.ds(g * GCH, GCH)]],
                             xg.at[pl.ds(g * GCH, GCH)], gsem)
            return 0

        lax.fori_loop(0, ng, _fg, 0)

    def _drain_gather(xg, ng):
        def _dg(g, _):
            pltpu.make_async_copy(rows.at[pl.ds(0, GCH)],
                                  xg.at[pl.ds(g * GCH, GCH)], gsem).wait()
            return 0

        lax.fori_loop(0, ng, _dg, 0)

    def _scale_scatter(b, xg, ng):
        def _sc(v, _):
            sl = pl.ds(v * 16, 16)
            xg[sl] = xg[sl] * wbuf[sl]
            return 0

        lax.fori_loop(0, ng * (GCH // 16), _sc, 0)

        def _fs(g, _):
            pltpu.async_copy(xg.at[pl.ds(g * GCH, GCH)],
                             accs[b].at[rbuf.at[pl.ds(g * GCH, GCH)]],
                             ssem, add=True)
            return 0

        lax.fori_loop(0, ng, _fs, 0)

    def _drain_scatter(xg, ng):
        def _ds(g, _):
            pltpu.make_async_copy(rows.at[pl.ds(0, GCH)],
                                  xg.at[pl.ds(g * GCH, GCH)], ssem).wait()
            return 0

        lax.fori_loop(0, ng, _ds, 0)

    def _flush(ng):
        _gather_plane(0, xgA, ng)
        for b in range(B):
            xg, xo = (xgA, xgB) if b % 2 == 0 else (xgB, xgA)
            _drain_gather(xg, ng)
            if b + 1 < B:
                if b >= 1:
                    _drain_scatter(xo, ng)
                _gather_plane(b + 1, xo, ng)
            _scale_scatter(b, xg, ng)
        _drain_scatter(xgA, ng)
        _drain_scatter(xgB, ng)

        def _zw(g, _):
            wbuf[pl.ds(g * 16, 16)] = zerov_f
            return 0

        lax.fori_loop(0, ng * (GCH // 16), _zw, 0)

    def _chunk(j, _):
        lo = (c * CPS + j) * CH

        for b in range(B):
            for z in range(ROWS_PT // ZR):
                pltpu.sync_copy(
                    zbuf, accs[b].at[pl.ds(s * ROWS_PT + z * ZR, ZR)])
        plsc.subcore_barrier()

        def _process(er, ec, ew, ptr0):
            def _grp(g, ptr):
                sl = pl.ds(g * 16, 16)
                local = er[sl] - lo
                m = local.astype(jnp.uint32) < jnp.uint32(CH)
                dsl = pl.ds(ptr, 16)
                plsc.store_compressed(cbuf.at[dsl], ec[sl], mask=m)
                plsc.store_compressed(rbuf.at[dsl], local, mask=m)
                plsc.store_compressed(wbuf.at[dsl], ew[sl], mask=m)
                p2 = ptr + plsc.all_reduce_population_count(m)[0]
                full = p2 > K - 16
                pl.when(full)(lambda: _flush(jnp.int32(GQ)))
                return jnp.where(full, 0, p2)

            return lax.fori_loop(0, BLK // 16, _grp, ptr0)

        ebase = s * EPT
        _fire(ebase, er0, ec0, ew0, esemA)

        def _blkpair(b2, ptr):
            _fire(ebase + (2 * b2 + 1) * BLK, er1, ec1, ew1, esemB)
            _wait(er0, ec0, ew0, esemA)
            ptr = _process(er0, ec0, ew0, ptr)
            _fire(ebase + ((2 * b2 + 2) % NBLK) * BLK, er0, ec0, ew0,
                  esemA)
            _wait(er1, ec1, ew1, esemB)
            ptr = _process(er1, ec1, ew1, ptr)
            return ptr

        ptr_end = lax.fori_loop(0, NBLK // 2, _blkpair, 0)
        _wait(er0, ec0, ew0, esemA)
        _flush((ptr_end + GCH - 1) // GCH)

        plsc.subcore_barrier()
        for b in range(B):
            pltpu.sync_copy(accs[b].at[pl.ds(s * ROWS_PT, ROWS_PT)],
                            outs[b].at[pl.ds(lo + s * ROWS_PT, ROWS_PT)])
        return 0

    lax.fori_loop(0, CPS, _chunk, 0)


def _sc_call(xplanes, rows, cols, wvs):
    mesh = plsc.VectorSubcoreMesh(core_axis_name="c", subcore_axis_name="s")
    kern = pl.kernel(
        _sc_body,
        out_type=[jax.ShapeDtypeStruct((R,), _f32) for _ in range(B)],
        mesh=mesh,
        scratch_types=[
            pltpu.VMEM((BLK,), _i32), pltpu.VMEM((BLK,), _i32),
            pltpu.VMEM((BLK,), _f32),
            pltpu.VMEM((BLK,), _i32), pltpu.VMEM((BLK,), _i32),
            pltpu.VMEM((BLK,), _f32),
            pltpu.VMEM((KC,), _i32),
            pltpu.VMEM((KC,), _i32),
            pltpu.VMEM((KC,), _f32),
            pltpu.VMEM((KC,), _f32),
            pltpu.VMEM((KC,), _f32),
            pltpu.VMEM((ZR,), _f32),
        ] + [pltpu.VMEM_SHARED((CH,), _f32) for _ in range(B)] + [
            pltpu.SemaphoreType.DMA, pltpu.SemaphoreType.DMA,
            pltpu.SemaphoreType.DMA, pltpu.SemaphoreType.DMA,
        ],
        compiler_params=pltpu.CompilerParams(needs_layout_passes=False,
                                             use_tc_tiling_on_sc=False),
    )
    return kern(*xplanes, rows, cols, wvs)


@jax.jit
def kernel(input, signal, edge_type, weight):
    inp2 = input.reshape(B, R)
    xplanes = [inp2[b] for b in range(B)]
    signal3 = signal.reshape(2, NNZ // _PC, _PC)
    et2 = edge_type.reshape(NNZ // _PC, _PC)
    rp, cp, wp = _prep(weight, signal3, et2)
    outs = _sc_call(xplanes, rp.reshape(-1), cp.reshape(-1), wp.reshape(-1))
    y = jnp.stack(outs, axis=0)
    return y.reshape(B, SITES, OUT_F)
